# baseline (speedup 1.0000x reference)
import jax
import jax.numpy as jnp
from jax.experimental import pallas as pl
from jax.experimental.pallas import tpu as pltpu

_N = 10000
_F_IN = 128
_F_H = 32
_TILE = 2000


def _fused_kernel(x_ref, wz_ref, wh_ref, bz_ref, bh_ref, wlin_ref, blin_ref,
                  out_ref, acc_ref):
    i = pl.program_id(0)
    wz = wz_ref[0, 0, :_F_IN, :] + wz_ref[1, 0, :_F_IN, :]
    wh = wh_ref[0, 0, :_F_IN, :] + wh_ref[1, 0, :_F_IN, :]
    x = x_ref[...]
    a = jnp.dot(x, wz, preferred_element_type=jnp.float32) + bz_ref[...]
    b = jnp.dot(x, wh, preferred_element_type=jnp.float32) + bh_ref[...]
    one_minus_z = 0.5 * (1.0 - jnp.tanh(0.5 * a))
    h = jnp.maximum(one_minus_z * jnp.tanh(b), 0.0)
    colsum = jnp.sum(h, axis=0, keepdims=True)

    @pl.when(i == 0)
    def _init():
        acc_ref[...] = jnp.zeros_like(acc_ref)

    acc_ref[...] += colsum

    @pl.when(i == pl.num_programs(0) - 1)
    def _finish():
        out_ref[...] = (jnp.sum(acc_ref[...] * wlin_ref[...], keepdims=True)
                        / _N + blin_ref[...])


def kernel(x, edge_index, edge_weight, W_z, b_z, W_r, b_r, W_h, b_h,
           W_lin, b_lin):
    del edge_index, edge_weight, W_r, b_r
    grid = (_N // _TILE,)
    return pl.pallas_call(
        _fused_kernel,
        grid=grid,
        in_specs=[
            pl.BlockSpec((_TILE, _F_IN), lambda i: (i, 0)),
            pl.BlockSpec((2, 1, _F_IN + _F_H, _F_H), lambda i: (0, 0, 0, 0)),
            pl.BlockSpec((2, 1, _F_IN + _F_H, _F_H), lambda i: (0, 0, 0, 0)),
            pl.BlockSpec((1, _F_H), lambda i: (0, 0)),
            pl.BlockSpec((1, _F_H), lambda i: (0, 0)),
            pl.BlockSpec((1, _F_H), lambda i: (0, 0)),
            pl.BlockSpec((1, 1), lambda i: (0, 0)),
        ],
        out_specs=pl.BlockSpec((1, 1), lambda i: (0, 0)),
        out_shape=jax.ShapeDtypeStruct((1, 1), jnp.float32),
        scratch_shapes=[pltpu.VMEM((1, _F_H), jnp.float32)],
    )(x, W_z, W_h, b_z.reshape(1, _F_H), b_h.reshape(1, _F_H),
      W_lin, b_lin.reshape(1, 1))


# speedup vs baseline: 1.0654x; 1.0654x over previous
import jax
import jax.numpy as jnp
from jax.experimental import pallas as pl
from jax.experimental.pallas import tpu as pltpu

_N = 10000
_F_IN = 128
_F_H = 32
_TILE = 5000


def _fused_kernel(x_ref, wz_ref, wh_ref, bz_ref, bh_ref, wlin_ref, blin_ref,
                  out_ref, acc_ref):
    i = pl.program_id(0)
    wz = wz_ref[0, 0, :_F_IN, :] + wz_ref[1, 0, :_F_IN, :]
    wh = wh_ref[0, 0, :_F_IN, :] + wh_ref[1, 0, :_F_IN, :]
    x = x_ref[...]
    a = jnp.dot(x, wz, preferred_element_type=jnp.float32) + bz_ref[...]
    b = jnp.dot(x, wh, preferred_element_type=jnp.float32) + bh_ref[...]
    one_minus_z = 0.5 * (1.0 - jnp.tanh(0.5 * a))
    h = jnp.maximum(one_minus_z * jnp.tanh(b), 0.0)
    colsum = jnp.sum(h, axis=0, keepdims=True)

    @pl.when(i == 0)
    def _init():
        acc_ref[...] = jnp.zeros_like(acc_ref)

    acc_ref[...] += colsum

    @pl.when(i == pl.num_programs(0) - 1)
    def _finish():
        out_ref[...] = (jnp.sum(acc_ref[...] * wlin_ref[...], keepdims=True)
                        / _N + blin_ref[...])


def kernel(x, edge_index, edge_weight, W_z, b_z, W_r, b_r, W_h, b_h,
           W_lin, b_lin):
    del edge_index, edge_weight, W_r, b_r
    grid = (_N // _TILE,)
    return pl.pallas_call(
        _fused_kernel,
        grid=grid,
        in_specs=[
            pl.BlockSpec((_TILE, _F_IN), lambda i: (i, 0)),
            pl.BlockSpec((2, 1, _F_IN + _F_H, _F_H), lambda i: (0, 0, 0, 0)),
            pl.BlockSpec((2, 1, _F_IN + _F_H, _F_H), lambda i: (0, 0, 0, 0)),
            pl.BlockSpec((1, _F_H), lambda i: (0, 0)),
            pl.BlockSpec((1, _F_H), lambda i: (0, 0)),
            pl.BlockSpec((1, _F_H), lambda i: (0, 0)),
            pl.BlockSpec((1, 1), lambda i: (0, 0)),
        ],
        out_specs=pl.BlockSpec((1, 1), lambda i: (0, 0)),
        out_shape=jax.ShapeDtypeStruct((1, 1), jnp.float32),
        scratch_shapes=[pltpu.VMEM((1, _F_H), jnp.float32)],
    )(x, W_z, W_h, b_z.reshape(1, _F_H), b_h.reshape(1, _F_H),
      W_lin, b_lin.reshape(1, 1))


# speedup vs baseline: 1.1166x; 1.0481x over previous
import jax
import jax.numpy as jnp
from jax.experimental import pallas as pl

_N = 10000
_F_IN = 128
_F_H = 32


def _fused_kernel(x_ref, wz_ref, wh_ref, bz_ref, bh_ref, wlin_ref, blin_ref,
                  out_ref):
    wz = wz_ref[0, 0, :_F_IN, :] + wz_ref[1, 0, :_F_IN, :]
    wh = wh_ref[0, 0, :_F_IN, :] + wh_ref[1, 0, :_F_IN, :]
    x = x_ref[...]
    a = jnp.dot(x, wz, preferred_element_type=jnp.float32) + bz_ref[...]
    b = jnp.dot(x, wh, preferred_element_type=jnp.float32) + bh_ref[...]
    one_minus_z = 0.5 * (1.0 - jnp.tanh(0.5 * a))
    h = jnp.maximum(one_minus_z * jnp.tanh(b), 0.0)
    colsum = jnp.sum(h, axis=0, keepdims=True)
    out_ref[...] = (jnp.sum(colsum * wlin_ref[...], keepdims=True) / _N
                    + blin_ref[...])


def kernel(x, edge_index, edge_weight, W_z, b_z, W_r, b_r, W_h, b_h,
           W_lin, b_lin):
    del edge_index, edge_weight, W_r, b_r
    return pl.pallas_call(
        _fused_kernel,
        out_shape=jax.ShapeDtypeStruct((1, 1), jnp.float32),
    )(x, W_z, W_h, b_z.reshape(1, _F_H), b_h.reshape(1, _F_H),
      W_lin, b_lin.reshape(1, 1))
